# Initial kernel scaffold; baseline (speedup 1.0000x reference)
import jax
import jax.numpy as jnp
from jax.experimental import pallas as pl


def kernel(x, positions, q_values, params, k_spatial, k_q):
    raise NotImplementedError("write your pallas kernel here")


# speedup vs baseline: 1.0338x; 1.0338x over previous
import functools

import jax
import jax.numpy as jnp
from jax import lax
from jax.experimental import pallas as pl
from jax.experimental.pallas import tpu as pltpu

B, N, NODE_DIM, HID, PROJ = 8, 2048, 5, 128, 256
NUM_MP = 3
BOX = 1.0
K_TOTAL = 48
NT = B * N


def _encoder_body(x_ref, w0_ref, b0_ref, w1_ref, b1_ref, h_ref):
    x = x_ref[...]
    h = jnp.maximum(jnp.dot(x, w0_ref[...], preferred_element_type=jnp.float32)
                    + b0_ref[...], 0.0)
    h = jnp.maximum(jnp.dot(h, w1_ref[...], preferred_element_type=jnp.float32)
                    + b1_ref[...], 0.0)
    h_ref[...] = h


def _encode(x_pad, w0p, b0, w1, b1):
    blk = 2048
    grid = (NT // blk,)
    return pl.pallas_call(
        _encoder_body,
        grid=grid,
        in_specs=[
            pl.BlockSpec((blk, 8), lambda i: (i, 0)),
            pl.BlockSpec((8, HID), lambda i: (0, 0)),
            pl.BlockSpec((1, HID), lambda i: (0, 0)),
            pl.BlockSpec((HID, HID), lambda i: (0, 0)),
            pl.BlockSpec((1, HID), lambda i: (0, 0)),
        ],
        out_specs=pl.BlockSpec((blk, HID), lambda i: (i, 0)),
        out_shape=jax.ShapeDtypeStruct((NT, HID), jnp.float32),
    )(x_pad, w0p, b0, w1, b1)


def _build_edges(positions, q_values):
    def per_batch(pos, q):
        delta = jnp.abs(pos[:, None, :] - pos[None, :, :])
        delta = jnp.where(delta > 0.5 * BOX, BOX - delta, delta)
        dist = jnp.sqrt((delta ** 2).sum(-1) + 1e-12)
        qn = q / jnp.maximum(jnp.linalg.norm(q, axis=-1, keepdims=True), 1e-8)
        q_dist = 1.0 - qn @ qn.T
        score = dist - 0.5 * q_dist
        score = jnp.where(jnp.eye(N, dtype=bool), jnp.inf, score)
        _, idx = jax.lax.top_k(-score, K_TOTAL)
        return idx

    topk = jax.vmap(per_batch)(positions, q_values)
    dst = topk.reshape(B, -1) + (jnp.arange(B) * N)[:, None]
    return dst.reshape(-1)


def kernel(x, positions, q_values, params, k_spatial, k_q):
    p = params
    x_pad = jnp.pad(x.reshape(NT, NODE_DIM), ((0, 0), (0, 8 - NODE_DIM)))
    w0p = jnp.pad(p['enc_W0'], ((0, 8 - NODE_DIM), (0, 0)))

    h = _encode(x_pad, w0p, p['enc_b0'][None, :], p['enc_W1'], p['enc_b1'][None, :])

    dst = _build_edges(positions, q_values)
    src = jnp.arange(NT, dtype=jnp.int32).repeat(K_TOTAL)

    deg = jnp.zeros((NT,), jnp.float32).at[dst].add(1.0)
    deg = jnp.maximum(deg, 1.0)

    for i in range(NUM_MP):
        wm, bm = p['msg_W%d' % i], p['msg_b%d' % i]
        wu, bu = p['upd_W%d' % i], p['upd_b%d' % i]
        a_rows = h @ wm[:HID] + bm
        b_rows = h @ wm[HID:]
        msg = jnp.maximum(a_rows[src] + b_rows[dst], 0.0)
        agg = jnp.zeros((NT, HID), jnp.float32).at[dst].add(msg)
        m = agg / deg[:, None]
        h = h + jnp.maximum(h @ wu[:HID] + m @ wu[HID:] + bu, 0.0)

    z = jnp.maximum(h @ p['proj_W0'] + p['proj_b0'], 0.0) @ p['proj_W1'] + p['proj_b1']
    return h, z


# speedup vs baseline: 3.5471x; 3.4309x over previous
import functools

import jax
import jax.numpy as jnp
from jax import lax
from jax.experimental import pallas as pl
from jax.experimental.pallas import tpu as pltpu
from jax.experimental.pallas import tpu_sc as plsc

B, N, NODE_DIM, HID, PROJ = 8, 2048, 5, 128, 256
NUM_MP = 3
BOX = 1.0
K_TOTAL = 48
NT = B * N


_NC, _NS = 2, 16
_NW = _NC * _NS
_NODES_W = NT // _NW
_GN = 8
_GE = _GN * K_TOTAL
_NGROUPS = _NODES_W // _GN
_CORE_NODES = NT // _NC


def _edge_body(a_hbm, b_hbm, dst_hbm, out_hbm, agg_sh, a_buf, rows, idx_buf,
               idx_loc, sem):
    c = lax.axis_index("c")
    s = lax.axis_index("s")
    wid = c * _NS + s
    core_base = c * _CORE_NODES

    def _zrow(i, _):
        for d in range(HID // 16):
            rows[i, pl.ds(d * 16, 16)] = jnp.zeros((16,), jnp.float32)
        return 0
    lax.fori_loop(0, _GE, _zrow, 0)
    sh_base = s * _NODES_W
    pltpu.sync_copy(rows, agg_sh.at[pl.ds(sh_base, _GE)])
    pltpu.sync_copy(rows.at[pl.ds(0, _NODES_W - _GE)],
                    agg_sh.at[pl.ds(sh_base + _GE, _NODES_W - _GE)])
    plsc.subcore_barrier()

    def _group(g, _):
        edge_base = wid * (_NODES_W * K_TOTAL) + g * _GE
        node_base = wid * _NODES_W + g * _GN
        pltpu.sync_copy(dst_hbm.at[pl.ds(edge_base, _GE)], idx_buf)
        pltpu.sync_copy(a_hbm.at[pl.ds(node_base, _GN)], a_buf)

        def _loc(i, _):
            v = idx_buf[pl.ds(i * 16, 16)]
            idx_loc[pl.ds(i * 16, 16)] = v - core_base
            return 0
        lax.fori_loop(0, _GE // 16, _loc, 0)

        pltpu.async_copy(b_hbm.at[idx_buf], rows, sem).wait()

        for n in range(_GN):
            a_regs = [a_buf[n, pl.ds(d * 16, 16)] for d in range(HID // 16)]

            def _edge(e, _):
                r = n * K_TOTAL + e
                for d in range(HID // 16):
                    v = rows[r, pl.ds(d * 16, 16)] + a_regs[d]
                    rows[r, pl.ds(d * 16, 16)] = jnp.maximum(v, 0.0)
                return 0
            lax.fori_loop(0, K_TOTAL, _edge, 0)

        pltpu.sync_copy(rows, agg_sh.at[idx_loc], add=True)
        return 0

    lax.fori_loop(0, _NGROUPS, _group, 0)
    plsc.subcore_barrier()

    pltpu.sync_copy(agg_sh.at[pl.ds(sh_base, _NODES_W)],
                    out_hbm.at[pl.ds(wid * _NODES_W, _NODES_W)])


@jax.jit
def _edge_aggregate(a_rows, b_rows, dst_flat):
    mesh = plsc.VectorSubcoreMesh(core_axis_name="c", subcore_axis_name="s")
    return pl.kernel(
        _edge_body,
        out_type=jax.ShapeDtypeStruct((NT, HID), jnp.float32),
        mesh=mesh,
        scratch_types=[
            pltpu.VMEM_SHARED((_CORE_NODES, HID), jnp.float32),
            pltpu.VMEM((_GN, HID), jnp.float32),
            pltpu.VMEM((_GE, HID), jnp.float32),
            pltpu.VMEM((_GE,), jnp.int32),
            pltpu.VMEM((_GE,), jnp.int32),
            pltpu.SemaphoreType.DMA,
        ],
    )(a_rows, b_rows, dst_flat)


def _encoder_body(x_ref, w0_ref, b0_ref, w1_ref, b1_ref, h_ref):
    x = x_ref[...]
    h = jnp.maximum(jnp.dot(x, w0_ref[...], preferred_element_type=jnp.float32)
                    + b0_ref[...], 0.0)
    h = jnp.maximum(jnp.dot(h, w1_ref[...], preferred_element_type=jnp.float32)
                    + b1_ref[...], 0.0)
    h_ref[...] = h


def _encode(x_pad, w0p, b0, w1, b1):
    blk = 2048
    grid = (NT // blk,)
    return pl.pallas_call(
        _encoder_body,
        grid=grid,
        in_specs=[
            pl.BlockSpec((blk, 8), lambda i: (i, 0)),
            pl.BlockSpec((8, HID), lambda i: (0, 0)),
            pl.BlockSpec((1, HID), lambda i: (0, 0)),
            pl.BlockSpec((HID, HID), lambda i: (0, 0)),
            pl.BlockSpec((1, HID), lambda i: (0, 0)),
        ],
        out_specs=pl.BlockSpec((blk, HID), lambda i: (i, 0)),
        out_shape=jax.ShapeDtypeStruct((NT, HID), jnp.float32),
    )(x_pad, w0p, b0, w1, b1)


def _build_edges(positions, q_values):
    def per_batch(pos, q):
        delta = jnp.abs(pos[:, None, :] - pos[None, :, :])
        delta = jnp.where(delta > 0.5 * BOX, BOX - delta, delta)
        dist = jnp.sqrt((delta ** 2).sum(-1) + 1e-12)
        qn = q / jnp.maximum(jnp.linalg.norm(q, axis=-1, keepdims=True), 1e-8)
        q_dist = 1.0 - qn @ qn.T
        score = dist - 0.5 * q_dist
        score = jnp.where(jnp.eye(N, dtype=bool), jnp.inf, score)
        _, idx = jax.lax.top_k(-score, K_TOTAL)
        return idx

    topk = jax.vmap(per_batch)(positions, q_values)
    dst = topk.reshape(B, -1) + (jnp.arange(B) * N)[:, None]
    return dst.reshape(-1)


def kernel(x, positions, q_values, params, k_spatial, k_q):
    p = params
    x_pad = jnp.pad(x.reshape(NT, NODE_DIM), ((0, 0), (0, 8 - NODE_DIM)))
    w0p = jnp.pad(p['enc_W0'], ((0, 8 - NODE_DIM), (0, 0)))

    h = _encode(x_pad, w0p, p['enc_b0'][None, :], p['enc_W1'], p['enc_b1'][None, :])

    dst = _build_edges(positions, q_values)

    deg = jnp.zeros((NT,), jnp.float32).at[dst].add(1.0)
    deg = jnp.maximum(deg, 1.0)

    for i in range(NUM_MP):
        wm, bm = p['msg_W%d' % i], p['msg_b%d' % i]
        wu, bu = p['upd_W%d' % i], p['upd_b%d' % i]
        a_rows = h @ wm[:HID] + bm
        b_rows = h @ wm[HID:]
        agg = _edge_aggregate(a_rows, b_rows, dst)
        m = agg / deg[:, None]
        h = h + jnp.maximum(h @ wu[:HID] + m @ wu[HID:] + bu, 0.0)

    z = jnp.maximum(h @ p['proj_W0'] + p['proj_b0'], 0.0) @ p['proj_W1'] + p['proj_b1']
    return h, z


# speedup vs baseline: 3.5479x; 1.0002x over previous
import functools

import jax
import jax.numpy as jnp
from jax import lax
from jax.experimental import pallas as pl
from jax.experimental.pallas import tpu as pltpu
from jax.experimental.pallas import tpu_sc as plsc

B, N, NODE_DIM, HID, PROJ = 8, 2048, 5, 128, 256
NUM_MP = 3
BOX = 1.0
K_TOTAL = 48
NT = B * N


_NC, _NS = 2, 16
_NW = _NC * _NS
_NODES_W = NT // _NW
_GN = 8
_GE = _GN * K_TOTAL
_NGROUPS = _NODES_W // _GN
_CORE_NODES = NT // _NC


def _edge_body(a_hbm, b_hbm, dst_hbm, out_hbm, agg_sh, a_buf, rows, idx_buf,
               idx_loc, sem):
    c = lax.axis_index("c")
    s = lax.axis_index("s")
    wid = c * _NS + s
    core_base = c * _CORE_NODES

    def _zrow(i, _):
        for d in range(HID // 16):
            rows[i, pl.ds(d * 16, 16)] = jnp.zeros((16,), jnp.float32)
        return 0
    lax.fori_loop(0, _GE, _zrow, 0)
    sh_base = s * _NODES_W
    pltpu.sync_copy(rows, agg_sh.at[pl.ds(sh_base, _GE)])
    pltpu.sync_copy(rows.at[pl.ds(0, _NODES_W - _GE)],
                    agg_sh.at[pl.ds(sh_base + _GE, _NODES_W - _GE)])
    plsc.subcore_barrier()

    def _group(g, _):
        edge_base = wid * (_NODES_W * K_TOTAL) + g * _GE
        node_base = wid * _NODES_W + g * _GN
        pltpu.sync_copy(dst_hbm.at[pl.ds(edge_base, _GE)], idx_buf)
        pltpu.sync_copy(a_hbm.at[pl.ds(node_base, _GN)], a_buf)

        def _loc(i, _):
            v = idx_buf[pl.ds(i * 16, 16)]
            idx_loc[pl.ds(i * 16, 16)] = v - core_base
            return 0
        lax.fori_loop(0, _GE // 16, _loc, 0)

        pltpu.async_copy(b_hbm.at[idx_buf], rows, sem).wait()

        for n in range(_GN):
            a_regs = [a_buf[n, pl.ds(d * 16, 16)] for d in range(HID // 16)]

            def _edge(e, _):
                r = n * K_TOTAL + e
                for d in range(HID // 16):
                    v = rows[r, pl.ds(d * 16, 16)] + a_regs[d]
                    rows[r, pl.ds(d * 16, 16)] = jnp.maximum(v, 0.0)
                return 0
            lax.fori_loop(0, K_TOTAL, _edge, 0)

        pltpu.sync_copy(rows, agg_sh.at[idx_loc], add=True)
        return 0

    lax.fori_loop(0, _NGROUPS, _group, 0)
    plsc.subcore_barrier()

    pltpu.sync_copy(agg_sh.at[pl.ds(sh_base, _NODES_W)],
                    out_hbm.at[pl.ds(wid * _NODES_W, _NODES_W)])


@jax.jit
def _edge_aggregate(a_rows, b_rows, dst_flat):
    mesh = plsc.VectorSubcoreMesh(core_axis_name="c", subcore_axis_name="s")
    return pl.kernel(
        _edge_body,
        out_type=jax.ShapeDtypeStruct((NT, HID), jnp.float32),
        mesh=mesh,
        scratch_types=[
            pltpu.VMEM_SHARED((_CORE_NODES, HID), jnp.float32),
            pltpu.VMEM((_GN, HID), jnp.float32),
            pltpu.VMEM((_GE, HID), jnp.float32),
            pltpu.VMEM((_GE,), jnp.int32),
            pltpu.VMEM((_GE,), jnp.int32),
            pltpu.SemaphoreType.DMA,
        ],
    )(a_rows, b_rows, dst_flat)


def _encoder_body(x_ref, w0_ref, b0_ref, w1_ref, b1_ref, h_ref):
    x = x_ref[...]
    h = jnp.maximum(jnp.dot(x, w0_ref[...], preferred_element_type=jnp.float32)
                    + b0_ref[...], 0.0)
    h = jnp.maximum(jnp.dot(h, w1_ref[...], preferred_element_type=jnp.float32)
                    + b1_ref[...], 0.0)
    h_ref[...] = h


def _encode(x_pad, w0p, b0, w1, b1):
    blk = 2048
    grid = (NT // blk,)
    return pl.pallas_call(
        _encoder_body,
        grid=grid,
        in_specs=[
            pl.BlockSpec((blk, 8), lambda i: (i, 0)),
            pl.BlockSpec((8, HID), lambda i: (0, 0)),
            pl.BlockSpec((1, HID), lambda i: (0, 0)),
            pl.BlockSpec((HID, HID), lambda i: (0, 0)),
            pl.BlockSpec((1, HID), lambda i: (0, 0)),
        ],
        out_specs=pl.BlockSpec((blk, HID), lambda i: (i, 0)),
        out_shape=jax.ShapeDtypeStruct((NT, HID), jnp.float32),
    )(x_pad, w0p, b0, w1, b1)


_RB = 256
_NB = N // _RB
_W16 = N // 16


def _knn_body(pos_r_ref, pos_c_ref, q_r_ref, q_c_ref, pk_ref,
              bits_ref, deg_ref):
    nb = pl.program_id(1)
    acc = jnp.zeros((_RB, N), jnp.float32)
    for d in range(3):
        pr = pos_r_ref[0, :, d:d + 1]
        pc = pos_c_ref[0, d:d + 1, :]
        delta = jnp.abs(pr - pc)
        delta = jnp.where(delta > 0.5 * BOX, BOX - delta, delta)
        acc = acc + delta * delta
    dist = jnp.sqrt(acc + 1e-12)
    qr0, qr1 = q_r_ref[0, :, 0:1], q_r_ref[0, :, 1:2]
    qc0, qc1 = q_c_ref[0, 0:1, :], q_c_ref[0, 1:2, :]
    nr = jnp.maximum(jnp.sqrt(qr0 * qr0 + qr1 * qr1), 1e-8)
    nc = jnp.maximum(jnp.sqrt(qc0 * qc0 + qc1 * qc1), 1e-8)
    qdot = (qr0 * qc0 + qr1 * qc1) / (nr * nc)
    score = dist - 0.5 * (1.0 - qdot)
    col = lax.broadcasted_iota(jnp.int32, (_RB, N), 1)
    row = lax.broadcasted_iota(jnp.int32, (_RB, N), 0) + nb * _RB
    score = jnp.where(col == row, jnp.inf, score)
    k32 = lax.bitcast_convert_type(score, jnp.int32)
    key = jnp.where(k32 < 0, k32 ^ jnp.int32(0x7FFFFFFF), k32)
    ukey = lax.bitcast_convert_type(key, jnp.uint32) ^ jnp.uint32(0x80000000)
    lo0 = jnp.zeros((_RB, 1), jnp.uint32)
    hi0 = jnp.full((_RB, 1), 0xFFFFFFFF, jnp.uint32)

    def _bs(_, carry):
        lo, hi = carry
        mid = lo + ((hi - lo) >> jnp.uint32(1))
        cnt = jnp.sum((ukey <= mid).astype(jnp.float32), axis=1,
                      keepdims=True)
        ge = cnt >= float(K_TOTAL)
        return jnp.where(ge, lo, mid + jnp.uint32(1)), jnp.where(ge, mid, hi)

    lo, hi = lax.fori_loop(0, 32, _bs, (lo0, hi0))
    mask = (ukey <= hi).astype(jnp.float32)
    deg_part = jnp.sum(mask, axis=0).reshape(1, 1, N)

    @pl.when(nb == 0)
    def _():
        deg_ref[...] = deg_part

    @pl.when(nb != 0)
    def _():
        deg_ref[...] += deg_part

    packed = jnp.dot(mask, pk_ref[...], preferred_element_type=jnp.float32)
    bits_ref[0] = packed.astype(jnp.int32)


@jax.jit
def _knn(pos_r, pos_c, q_r, q_c, pack_mat):
    grid = (B, _NB)
    bits, deg = pl.pallas_call(
        _knn_body,
        grid=grid,
        in_specs=[
            pl.BlockSpec((1, _RB, 3), lambda b, nb: (b, nb, 0)),
            pl.BlockSpec((1, 3, N), lambda b, nb: (b, 0, 0)),
            pl.BlockSpec((1, _RB, 2), lambda b, nb: (b, nb, 0)),
            pl.BlockSpec((1, 2, N), lambda b, nb: (b, 0, 0)),
            pl.BlockSpec((N, _W16), lambda b, nb: (0, 0)),
        ],
        out_specs=[
            pl.BlockSpec((1, _RB, _W16), lambda b, nb: (b, nb, 0)),
            pl.BlockSpec((1, 1, N), lambda b, nb: (b, 0, 0)),
        ],
        out_shape=[
            jax.ShapeDtypeStruct((B, N, _W16), jnp.int32),
            jax.ShapeDtypeStruct((B, 1, N), jnp.float32),
        ],
    )(pos_r, pos_c, q_r, q_c, pack_mat)
    return bits, deg


_XROWS = 64
_XCH = _NODES_W // _XROWS
_OPAD = 64


def _extract_body(bits_hbm, dst_hbm, bits_buf, out_buf, iota16):
    c = lax.axis_index("c")
    s = lax.axis_index("s")
    wid = c * _NS + s
    b_off = (wid // 4) * N
    lane16 = iota16[...]

    def _chunk(ch, _):
        row0 = wid * _NODES_W + ch * _XROWS
        pltpu.sync_copy(bits_hbm.at[pl.ds(row0, _XROWS)], bits_buf)

        def _row(r, _):
            ptr0 = jnp.broadcast_to(r * _OPAD, (16,)).astype(jnp.int32)
            ptr = ptr0
            for wv in range(_W16 // 16):
                w = bits_buf[r, pl.ds(wv * 16, 16)]
                base = lane16 + (b_off + wv * 256)
                for bit in range(16):
                    m = ((w >> bit) & 1) == 1
                    mi = m.astype(jnp.int32)
                    pos = plsc.cumsum(mi) - mi
                    plsc.store_scatter(out_buf, [ptr + pos], base + bit,
                                       mask=m)
                    ptr = ptr + plsc.all_reduce_population_count(m)
            return 0

        lax.fori_loop(0, _XROWS, _row, 0)
        pltpu.sync_copy(out_buf.at[pl.ds(0, _XROWS * _OPAD)],
                        dst_hbm.at[pl.ds(row0 * _OPAD, _XROWS * _OPAD)])
        return 0

    lax.fori_loop(0, _XCH, _chunk, 0)


@jax.jit
def _extract(bits):
    mesh = plsc.VectorSubcoreMesh(core_axis_name="c", subcore_axis_name="s")
    iota16 = jnp.arange(0, 256, 16, dtype=jnp.int32)
    return pl.kernel(
        _extract_body,
        out_type=jax.ShapeDtypeStruct((NT * _OPAD,), jnp.int32),
        mesh=mesh,
        scratch_types=[
            pltpu.VMEM((_XROWS, _W16), jnp.int32),
            pltpu.VMEM((_XROWS * _OPAD + 16,), jnp.int32),
        ],
    )(bits, iota16)


def _build_edges(positions, q_values):
    def per_batch(pos, q):
        delta = jnp.abs(pos[:, None, :] - pos[None, :, :])
        delta = jnp.where(delta > 0.5 * BOX, BOX - delta, delta)
        dist = jnp.sqrt((delta ** 2).sum(-1) + 1e-12)
        qn = q / jnp.maximum(jnp.linalg.norm(q, axis=-1, keepdims=True), 1e-8)
        q_dist = 1.0 - qn @ qn.T
        score = dist - 0.5 * q_dist
        score = jnp.where(jnp.eye(N, dtype=bool), jnp.inf, score)
        _, idx = lax.top_k(-score, K_TOTAL)
        return idx

    topk = jax.vmap(per_batch)(positions, q_values)
    dst = topk.reshape(B, -1) + (jnp.arange(B) * N)[:, None]
    return dst.reshape(-1)


def kernel(x, positions, q_values, params, k_spatial, k_q):
    p = params
    x_pad = jnp.pad(x.reshape(NT, NODE_DIM), ((0, 0), (0, 8 - NODE_DIM)))
    w0p = jnp.pad(p['enc_W0'], ((0, 8 - NODE_DIM), (0, 0)))

    h = _encode(x_pad, w0p, p['enc_b0'][None, :], p['enc_W1'], p['enc_b1'][None, :])

    dst = _build_edges(positions, q_values)

    deg = jnp.zeros((NT,), jnp.float32).at[dst].add(1.0)
    deg = jnp.maximum(deg, 1.0)

    for i in range(NUM_MP):
        wm, bm = p['msg_W%d' % i], p['msg_b%d' % i]
        wu, bu = p['upd_W%d' % i], p['upd_b%d' % i]
        a_rows = h @ wm[:HID] + bm
        b_rows = h @ wm[HID:]
        agg = _edge_aggregate(a_rows, b_rows, dst)
        m = agg / deg[:, None]
        h = h + jnp.maximum(h @ wu[:HID] + m @ wu[HID:] + bu, 0.0)

    z = jnp.maximum(h @ p['proj_W0'] + p['proj_b0'], 0.0) @ p['proj_W1'] + p['proj_b1']
    return h, z


# speedup vs baseline: 3.5527x; 1.0014x over previous
import functools

import jax
import jax.numpy as jnp
from jax import lax
from jax.experimental import pallas as pl
from jax.experimental.pallas import tpu as pltpu
from jax.experimental.pallas import tpu_sc as plsc

B, N, NODE_DIM, HID, PROJ = 8, 2048, 5, 128, 256
NUM_MP = 3
BOX = 1.0
K_TOTAL = 48
NT = B * N


_NC, _NS = 2, 16
_NW = _NC * _NS
_NODES_W = NT // _NW
_GN = 8
_GE = _GN * K_TOTAL
_NGROUPS = _NODES_W // _GN
_CORE_NODES = NT // _NC


def _edge_body(a_hbm, b_hbm, dst_hbm, out_hbm, agg_sh, a_buf, rows, idx_buf,
               idx_loc, sem):
    c = lax.axis_index("c")
    s = lax.axis_index("s")
    wid = c * _NS + s
    core_base = c * _CORE_NODES

    def _zrow(i, _):
        for d in range(HID // 16):
            rows[i, pl.ds(d * 16, 16)] = jnp.zeros((16,), jnp.float32)
        return 0
    lax.fori_loop(0, _GE, _zrow, 0)
    sh_base = s * _NODES_W
    pltpu.sync_copy(rows, agg_sh.at[pl.ds(sh_base, _GE)])
    pltpu.sync_copy(rows.at[pl.ds(0, _NODES_W - _GE)],
                    agg_sh.at[pl.ds(sh_base + _GE, _NODES_W - _GE)])
    plsc.subcore_barrier()

    def _group(g, _):
        edge_base = wid * (_NODES_W * K_TOTAL) + g * _GE
        node_base = wid * _NODES_W + g * _GN
        pltpu.sync_copy(dst_hbm.at[pl.ds(edge_base, _GE)], idx_buf)
        pltpu.sync_copy(a_hbm.at[pl.ds(node_base, _GN)], a_buf)

        def _loc(i, _):
            v = idx_buf[pl.ds(i * 16, 16)]
            idx_loc[pl.ds(i * 16, 16)] = v - core_base
            return 0
        lax.fori_loop(0, _GE // 16, _loc, 0)

        pltpu.async_copy(b_hbm.at[idx_buf], rows, sem).wait()

        for n in range(_GN):
            a_regs = [a_buf[n, pl.ds(d * 16, 16)] for d in range(HID // 16)]

            def _edge(e, _):
                r = n * K_TOTAL + e
                for d in range(HID // 16):
                    v = rows[r, pl.ds(d * 16, 16)] + a_regs[d]
                    rows[r, pl.ds(d * 16, 16)] = jnp.maximum(v, 0.0)
                return 0
            lax.fori_loop(0, K_TOTAL, _edge, 0)

        pltpu.sync_copy(rows, agg_sh.at[idx_loc], add=True)
        return 0

    lax.fori_loop(0, _NGROUPS, _group, 0)
    plsc.subcore_barrier()

    pltpu.sync_copy(agg_sh.at[pl.ds(sh_base, _NODES_W)],
                    out_hbm.at[pl.ds(wid * _NODES_W, _NODES_W)])


@jax.jit
def _edge_aggregate(a_rows, b_rows, dst_flat):
    mesh = plsc.VectorSubcoreMesh(core_axis_name="c", subcore_axis_name="s")
    return pl.kernel(
        _edge_body,
        out_type=jax.ShapeDtypeStruct((NT, HID), jnp.float32),
        mesh=mesh,
        scratch_types=[
            pltpu.VMEM_SHARED((_CORE_NODES, HID), jnp.float32),
            pltpu.VMEM((_GN, HID), jnp.float32),
            pltpu.VMEM((_GE, HID), jnp.float32),
            pltpu.VMEM((_GE,), jnp.int32),
            pltpu.VMEM((_GE,), jnp.int32),
            pltpu.SemaphoreType.DMA,
        ],
    )(a_rows, b_rows, dst_flat)


def _encoder_body(x_ref, w0_ref, b0_ref, w1_ref, b1_ref, h_ref):
    x = x_ref[...]
    h = jnp.maximum(jnp.dot(x, w0_ref[...], preferred_element_type=jnp.float32)
                    + b0_ref[...], 0.0)
    h = jnp.maximum(jnp.dot(h, w1_ref[...], preferred_element_type=jnp.float32)
                    + b1_ref[...], 0.0)
    h_ref[...] = h


def _encode(x_pad, w0p, b0, w1, b1):
    blk = 2048
    grid = (NT // blk,)
    return pl.pallas_call(
        _encoder_body,
        grid=grid,
        in_specs=[
            pl.BlockSpec((blk, 8), lambda i: (i, 0)),
            pl.BlockSpec((8, HID), lambda i: (0, 0)),
            pl.BlockSpec((1, HID), lambda i: (0, 0)),
            pl.BlockSpec((HID, HID), lambda i: (0, 0)),
            pl.BlockSpec((1, HID), lambda i: (0, 0)),
        ],
        out_specs=pl.BlockSpec((blk, HID), lambda i: (i, 0)),
        out_shape=jax.ShapeDtypeStruct((NT, HID), jnp.float32),
    )(x_pad, w0p, b0, w1, b1)


_RB = 256
_NB = N // _RB
_W16 = N // 16


def _knn_body(pos_r_ref, pos_c_ref, q_r_ref, q_c_ref, pk_ref,
              bits_ref, deg_ref):
    nb = pl.program_id(1)
    acc = jnp.zeros((_RB, N), jnp.float32)
    for d in range(3):
        pr = pos_r_ref[0, :, d:d + 1]
        pc = pos_c_ref[0, d:d + 1, :]
        delta = jnp.abs(pr - pc)
        delta = jnp.where(delta > 0.5 * BOX, BOX - delta, delta)
        acc = acc + delta * delta
    dist = jnp.sqrt(acc + 1e-12)
    qr0, qr1 = q_r_ref[0, :, 0:1], q_r_ref[0, :, 1:2]
    qc0, qc1 = q_c_ref[0, 0:1, :], q_c_ref[0, 1:2, :]
    nr = jnp.maximum(jnp.sqrt(qr0 * qr0 + qr1 * qr1), 1e-8)
    nc = jnp.maximum(jnp.sqrt(qc0 * qc0 + qc1 * qc1), 1e-8)
    qdot = (qr0 * qc0 + qr1 * qc1) / (nr * nc)
    score = dist - 0.5 * (1.0 - qdot)
    col = lax.broadcasted_iota(jnp.int32, (_RB, N), 1)
    row = lax.broadcasted_iota(jnp.int32, (_RB, N), 0) + nb * _RB
    score = jnp.where(col == row, jnp.inf, score)
    k32 = lax.bitcast_convert_type(score, jnp.int32)
    key = jnp.where(k32 < 0, k32 ^ jnp.int32(0x7FFFFFFF), k32)
    ukey = lax.bitcast_convert_type(key, jnp.uint32) ^ jnp.uint32(0x80000000)
    lo0 = jnp.zeros((_RB, 1), jnp.uint32)
    hi0 = jnp.full((_RB, 1), 0xFFFFFFFF, jnp.uint32)

    def _bs(_, carry):
        lo, hi = carry
        mid = lo + ((hi - lo) >> jnp.uint32(1))
        cnt = jnp.sum((ukey <= mid).astype(jnp.float32), axis=1,
                      keepdims=True)
        ge = cnt >= float(K_TOTAL)
        return jnp.where(ge, lo, mid + jnp.uint32(1)), jnp.where(ge, mid, hi)

    lo, hi = lax.fori_loop(0, 32, _bs, (lo0, hi0))
    mask = (ukey <= hi).astype(jnp.float32)
    deg_part = jnp.sum(mask, axis=0).reshape(1, 1, N)

    @pl.when(nb == 0)
    def _():
        deg_ref[...] = deg_part

    @pl.when(nb != 0)
    def _():
        deg_ref[...] += deg_part

    packed = jnp.dot(mask, pk_ref[...], preferred_element_type=jnp.float32)
    bits_ref[0] = packed.astype(jnp.int32)


@jax.jit
def _knn(pos_r, pos_c, q_r, q_c, pack_mat):
    grid = (B, _NB)
    bits, deg = pl.pallas_call(
        _knn_body,
        grid=grid,
        in_specs=[
            pl.BlockSpec((1, _RB, 3), lambda b, nb: (b, nb, 0)),
            pl.BlockSpec((1, 3, N), lambda b, nb: (b, 0, 0)),
            pl.BlockSpec((1, _RB, 2), lambda b, nb: (b, nb, 0)),
            pl.BlockSpec((1, 2, N), lambda b, nb: (b, 0, 0)),
            pl.BlockSpec((N, _W16), lambda b, nb: (0, 0)),
        ],
        out_specs=[
            pl.BlockSpec((1, _RB, _W16), lambda b, nb: (b, nb, 0)),
            pl.BlockSpec((1, 1, N), lambda b, nb: (b, 0, 0)),
        ],
        out_shape=[
            jax.ShapeDtypeStruct((B, N, _W16), jnp.int32),
            jax.ShapeDtypeStruct((B, 1, N), jnp.float32),
        ],
    )(pos_r, pos_c, q_r, q_c, pack_mat)
    return bits, deg


_XROWS = 64
_XCH = _NODES_W // _XROWS
_OPAD = K_TOTAL


def _extract_body(bits_hbm, dst_hbm, bits_buf, out_buf, pfx_buf):
    c = lax.axis_index("c")
    s = lax.axis_index("s")
    wid = c * _NS + s
    b_off = (wid // 4) * N
    iot = lax.iota(jnp.int32, 16)
    lane16 = iot * 16
    gmask = [jnp.maximum(iot - k, 0) for k in (1, 2, 4, 8)]
    smask = [jnp.minimum(jnp.maximum(iot - (k - 1), 0), 1)
             for k in (1, 2, 4, 8)]
    l15 = jnp.broadcast_to(15, (16,)).astype(jnp.int32)

    def _chunk(ch, _):
        row0 = wid * _NODES_W + ch * _XROWS
        pltpu.sync_copy(bits_hbm.at[pl.ds(row0, _XROWS)], bits_buf)

        def _row(r, _):
            rowptr = jnp.broadcast_to(r * _OPAD, (16,)).astype(jnp.int32)
            w = bits_buf[r, pl.ds(0, 16)]
            mi = (w >> 1) & 1
            plsc.store_scatter(out_buf, [rowptr + iot], lane16 + mi,
                               mask=(mi == 1))
            return 0

        lax.fori_loop(0, _XROWS, _row, 0)
        pltpu.sync_copy(out_buf.at[pl.ds(0, _XROWS * _OPAD)],
                        dst_hbm.at[pl.ds(row0 * _OPAD, _XROWS * _OPAD)])
        return 0

    lax.fori_loop(0, _XCH, _chunk, 0)


@jax.jit
def _extract(bits):
    mesh = plsc.VectorSubcoreMesh(core_axis_name="c", subcore_axis_name="s")
    return pl.kernel(
        _extract_body,
        out_type=jax.ShapeDtypeStruct((NT * _OPAD,), jnp.int32),
        mesh=mesh,
        scratch_types=[
            pltpu.VMEM((_XROWS, _W16), jnp.int32),
            pltpu.VMEM((_XROWS * _OPAD + 16,), jnp.int32),
            pltpu.VMEM((16,), jnp.int32),
        ],
    )(bits)


def _build_edges(positions, q_values):
    delta = jnp.abs(positions[:, :, None, :] - positions[:, None, :, :])
    delta = jnp.where(delta > 0.5 * BOX, BOX - delta, delta)
    dist = jnp.sqrt((delta ** 2).sum(-1) + 1e-12)
    qn = q_values / jnp.maximum(
        jnp.linalg.norm(q_values, axis=-1, keepdims=True), 1e-8)
    q_dist = 1.0 - jnp.einsum('bnc,bmc->bnm', qn, qn)
    score = dist - 0.5 * q_dist
    score = jnp.where(jnp.eye(N, dtype=bool)[None], jnp.inf, score)
    _, idx = lax.top_k(-score, K_TOTAL)
    dst = (idx + (jnp.arange(B) * N)[:, None, None]).reshape(-1)
    deg = jnp.zeros((NT,), jnp.float32).at[dst].add(1.0)
    return dst, jnp.maximum(deg, 1.0)


def kernel(x, positions, q_values, params, k_spatial, k_q):
    p = params
    x_pad = jnp.pad(x.reshape(NT, NODE_DIM), ((0, 0), (0, 8 - NODE_DIM)))
    w0p = jnp.pad(p['enc_W0'], ((0, 8 - NODE_DIM), (0, 0)))

    h = _encode(x_pad, w0p, p['enc_b0'][None, :], p['enc_W1'], p['enc_b1'][None, :])

    dst, deg = _build_edges(positions, q_values)

    for i in range(NUM_MP):
        wm, bm = p['msg_W%d' % i], p['msg_b%d' % i]
        wu, bu = p['upd_W%d' % i], p['upd_b%d' % i]
        a_rows = h @ wm[:HID] + bm
        b_rows = h @ wm[HID:]
        agg = _edge_aggregate(a_rows, b_rows, dst)
        m = agg / deg[:, None]
        h = h + jnp.maximum(h @ wu[:HID] + m @ wu[HID:] + bu, 0.0)

    z = jnp.maximum(h @ p['proj_W0'] + p['proj_b0'], 0.0) @ p['proj_W1'] + p['proj_b1']
    return h, z


# speedup vs baseline: 7.1008x; 1.9987x over previous
import functools

import jax
import jax.numpy as jnp
from jax import lax
from jax.experimental import pallas as pl
from jax.experimental.pallas import tpu as pltpu
from jax.experimental.pallas import tpu_sc as plsc

B, N, NODE_DIM, HID, PROJ = 8, 2048, 5, 128, 256
NUM_MP = 3
BOX = 1.0
K_TOTAL = 48
NT = B * N


_NC, _NS = 2, 16
_NW = _NC * _NS
_NODES_W = NT // _NW
_GN = 8
_GE = _GN * K_TOTAL
_NGROUPS = _NODES_W // _GN
_CORE_NODES = NT // _NC


def _edge_body(a_hbm, b_hbm, dst_hbm, out_hbm, agg_sh, a_buf, rows, idx_buf,
               idx_loc, sem):
    c = lax.axis_index("c")
    s = lax.axis_index("s")
    wid = c * _NS + s
    core_base = c * _CORE_NODES

    def _zrow(i, _):
        for d in range(HID // 16):
            rows[i, pl.ds(d * 16, 16)] = jnp.zeros((16,), jnp.float32)
        return 0
    lax.fori_loop(0, _GE, _zrow, 0)
    sh_base = s * _NODES_W
    pltpu.sync_copy(rows, agg_sh.at[pl.ds(sh_base, _GE)])
    pltpu.sync_copy(rows.at[pl.ds(0, _NODES_W - _GE)],
                    agg_sh.at[pl.ds(sh_base + _GE, _NODES_W - _GE)])
    plsc.subcore_barrier()

    def _group(g, _):
        edge_base = wid * (_NODES_W * K_TOTAL) + g * _GE
        node_base = wid * _NODES_W + g * _GN
        pltpu.sync_copy(dst_hbm.at[pl.ds(edge_base, _GE)], idx_buf)
        pltpu.sync_copy(a_hbm.at[pl.ds(node_base, _GN)], a_buf)

        def _loc(i, _):
            v = idx_buf[pl.ds(i * 16, 16)]
            idx_loc[pl.ds(i * 16, 16)] = v - core_base
            return 0
        lax.fori_loop(0, _GE // 16, _loc, 0)

        pltpu.async_copy(b_hbm.at[idx_buf], rows, sem).wait()

        for n in range(_GN):
            a_regs = [a_buf[n, pl.ds(d * 16, 16)] for d in range(HID // 16)]

            def _edge(e, _):
                r = n * K_TOTAL + e
                for d in range(HID // 16):
                    v = rows[r, pl.ds(d * 16, 16)] + a_regs[d]
                    rows[r, pl.ds(d * 16, 16)] = jnp.maximum(v, 0.0)
                return 0
            lax.fori_loop(0, K_TOTAL, _edge, 0)

        pltpu.sync_copy(rows, agg_sh.at[idx_loc], add=True)
        return 0

    lax.fori_loop(0, _NGROUPS, _group, 0)
    plsc.subcore_barrier()

    pltpu.sync_copy(agg_sh.at[pl.ds(sh_base, _NODES_W)],
                    out_hbm.at[pl.ds(wid * _NODES_W, _NODES_W)])


@jax.jit
def _edge_aggregate(a_rows, b_rows, dst_flat):
    mesh = plsc.VectorSubcoreMesh(core_axis_name="c", subcore_axis_name="s")
    return pl.kernel(
        _edge_body,
        out_type=jax.ShapeDtypeStruct((NT, HID), jnp.float32),
        mesh=mesh,
        scratch_types=[
            pltpu.VMEM_SHARED((_CORE_NODES, HID), jnp.float32),
            pltpu.VMEM((_GN, HID), jnp.float32),
            pltpu.VMEM((_GE, HID), jnp.float32),
            pltpu.VMEM((_GE,), jnp.int32),
            pltpu.VMEM((_GE,), jnp.int32),
            pltpu.SemaphoreType.DMA,
        ],
    )(a_rows, b_rows, dst_flat)


def _encoder_body(x_ref, w0_ref, b0_ref, w1_ref, b1_ref, h_ref):
    x = x_ref[...]
    h = jnp.maximum(jnp.dot(x, w0_ref[...], preferred_element_type=jnp.float32)
                    + b0_ref[...], 0.0)
    h = jnp.maximum(jnp.dot(h, w1_ref[...], preferred_element_type=jnp.float32)
                    + b1_ref[...], 0.0)
    h_ref[...] = h


def _encode(x_pad, w0p, b0, w1, b1):
    blk = 2048
    grid = (NT // blk,)
    return pl.pallas_call(
        _encoder_body,
        grid=grid,
        in_specs=[
            pl.BlockSpec((blk, 8), lambda i: (i, 0)),
            pl.BlockSpec((8, HID), lambda i: (0, 0)),
            pl.BlockSpec((1, HID), lambda i: (0, 0)),
            pl.BlockSpec((HID, HID), lambda i: (0, 0)),
            pl.BlockSpec((1, HID), lambda i: (0, 0)),
        ],
        out_specs=pl.BlockSpec((blk, HID), lambda i: (i, 0)),
        out_shape=jax.ShapeDtypeStruct((NT, HID), jnp.float32),
    )(x_pad, w0p, b0, w1, b1)


_RB = 256
_NB = N // _RB
_W16 = N // 16


def _knn_body(pos_r_ref, pos_c_ref, q_r_ref, q_c_ref, pk_ref,
              bits_ref, deg_ref):
    nb = pl.program_id(1)
    acc = jnp.zeros((_RB, N), jnp.float32)
    for d in range(3):
        pr = pos_r_ref[0, :, d:d + 1]
        pc = pos_c_ref[0, d:d + 1, :]
        delta = jnp.abs(pr - pc)
        delta = jnp.where(delta > 0.5 * BOX, BOX - delta, delta)
        acc = acc + delta * delta
    dist = jnp.sqrt(acc + 1e-12)
    qr0, qr1 = q_r_ref[0, :, 0:1], q_r_ref[0, :, 1:2]
    qc0, qc1 = q_c_ref[0, 0:1, :], q_c_ref[0, 1:2, :]
    nr = jnp.maximum(jnp.sqrt(qr0 * qr0 + qr1 * qr1), 1e-8)
    nc = jnp.maximum(jnp.sqrt(qc0 * qc0 + qc1 * qc1), 1e-8)
    qdot = (qr0 * qc0 + qr1 * qc1) / (nr * nc)
    score = dist - 0.5 * (1.0 - qdot)
    col = lax.broadcasted_iota(jnp.int32, (_RB, N), 1)
    row = lax.broadcasted_iota(jnp.int32, (_RB, N), 0) + nb * _RB
    score = jnp.where(col == row, jnp.inf, score)
    k32 = lax.bitcast_convert_type(score, jnp.int32)
    key = jnp.where(k32 < 0, k32 ^ jnp.int32(0x7FFFFFFF), k32)
    ukey = lax.bitcast_convert_type(key, jnp.uint32) ^ jnp.uint32(0x80000000)
    lo0 = jnp.zeros((_RB, 1), jnp.uint32)
    hi0 = jnp.full((_RB, 1), 0xFFFFFFFF, jnp.uint32)

    def _bs(_, carry):
        lo, hi = carry
        mid = lo + ((hi - lo) >> jnp.uint32(1))
        cnt = jnp.sum((ukey <= mid).astype(jnp.float32), axis=1,
                      keepdims=True)
        ge = cnt >= float(K_TOTAL)
        return jnp.where(ge, lo, mid + jnp.uint32(1)), jnp.where(ge, mid, hi)

    lo, hi = lax.fori_loop(0, 32, _bs, (lo0, hi0))
    mask = (ukey <= hi).astype(jnp.float32)
    deg_part = jnp.sum(mask, axis=0).reshape(1, 1, N)

    @pl.when(nb == 0)
    def _():
        deg_ref[...] = deg_part

    @pl.when(nb != 0)
    def _():
        deg_ref[...] += deg_part

    packed = jnp.dot(mask, pk_ref[...], preferred_element_type=jnp.float32)
    bits_ref[0] = packed.astype(jnp.int32)


@jax.jit
def _knn(pos_r, pos_c, q_r, q_c, pack_mat):
    grid = (B, _NB)
    bits, deg = pl.pallas_call(
        _knn_body,
        grid=grid,
        in_specs=[
            pl.BlockSpec((1, _RB, 3), lambda b, nb: (b, nb, 0)),
            pl.BlockSpec((1, 3, N), lambda b, nb: (b, 0, 0)),
            pl.BlockSpec((1, _RB, 2), lambda b, nb: (b, nb, 0)),
            pl.BlockSpec((1, 2, N), lambda b, nb: (b, 0, 0)),
            pl.BlockSpec((N, _W16), lambda b, nb: (0, 0)),
        ],
        out_specs=[
            pl.BlockSpec((1, _RB, _W16), lambda b, nb: (b, nb, 0)),
            pl.BlockSpec((1, 1, N), lambda b, nb: (b, 0, 0)),
        ],
        out_shape=[
            jax.ShapeDtypeStruct((B, N, _W16), jnp.int32),
            jax.ShapeDtypeStruct((B, 1, N), jnp.float32),
        ],
    )(pos_r, pos_c, q_r, q_c, pack_mat)
    return bits, deg


_XROWS = 64
_XCH = _NODES_W // _XROWS
_OPAD = K_TOTAL


_PR = 4


def _extract_body(bits_hbm, dst_hbm, bits_buf, out_buf):
    c = lax.axis_index("c")
    s = lax.axis_index("s")
    wid = c * _NS + s
    b_off = (wid // 4) * N

    def _chunk(ch, _):
        row0 = wid * _NODES_W + ch * _XROWS
        pltpu.sync_copy(bits_hbm.at[pl.ds(row0, _XROWS)], bits_buf)

        def _row(r, _):
            def _wvec(wv, ptr):
                wvec = bits_buf[r, pl.ds(wv * 16, 16)]
                base = b_off + wv * 256
                for lane in range(16):
                    w0 = wvec[lane]
                    col0 = base + lane * 16
                    v = w0 - ((w0 >> 1) & 0x5555)
                    v = (v & 0x3333) + ((v >> 2) & 0x3333)
                    v = (v + (v >> 4)) & 0x0F0F
                    pc = (v + (v >> 8)) & 0x1F

                    def _step(t, carry, col0=col0):
                        w, p = carry
                        lsb = w & (0 - w)
                        e = (lax.bitcast_convert_type(
                            lsb.astype(jnp.float32), jnp.int32) >> 23) - 127
                        out_buf[pl.ds(p, 16)] = jnp.broadcast_to(
                            col0 + e, (16,)).astype(jnp.int32)
                        return (w ^ lsb, p + 1)

                    _, ptr = lax.fori_loop(0, pc, _step, (w0, ptr))
                return ptr

            lax.fori_loop(0, _W16 // 16, _wvec, r * _OPAD)
            return 0

        lax.fori_loop(0, _XROWS, _row, 0)
        pltpu.sync_copy(out_buf.at[pl.ds(0, _XROWS * _OPAD)],
                        dst_hbm.at[pl.ds(row0 * _OPAD, _XROWS * _OPAD)])
        return 0

    lax.fori_loop(0, _XCH, _chunk, 0)


@jax.jit
def _extract(bits):
    mesh = plsc.VectorSubcoreMesh(core_axis_name="c", subcore_axis_name="s")
    return pl.kernel(
        _extract_body,
        out_type=jax.ShapeDtypeStruct((NT * _OPAD,), jnp.int32),
        mesh=mesh,
        scratch_types=[
            pltpu.VMEM((_XROWS, _W16), jnp.int32),
            pltpu.VMEM((_XROWS * _OPAD + 16,), jnp.int32),
        ],
    )(bits)


def _build_edges(positions, q_values):
    pos_c = positions.transpose(0, 2, 1)
    q_c = q_values.transpose(0, 2, 1)
    j = jnp.arange(N)
    pack_mat = ((j[:, None] // 16 == jnp.arange(_W16)[None, :])
                .astype(jnp.float32)
                * jnp.exp2((j % 16).astype(jnp.float32))[:, None])
    bits, deg = _knn(positions, pos_c, q_values, q_c, pack_mat)
    dst = _extract(bits.reshape(NT, _W16))
    return dst, jnp.maximum(deg.reshape(NT), 1.0)


def kernel(x, positions, q_values, params, k_spatial, k_q):
    p = params
    x_pad = jnp.pad(x.reshape(NT, NODE_DIM), ((0, 0), (0, 8 - NODE_DIM)))
    w0p = jnp.pad(p['enc_W0'], ((0, 8 - NODE_DIM), (0, 0)))

    h = _encode(x_pad, w0p, p['enc_b0'][None, :], p['enc_W1'], p['enc_b1'][None, :])

    dst, deg = _build_edges(positions, q_values)

    for i in range(NUM_MP):
        wm, bm = p['msg_W%d' % i], p['msg_b%d' % i]
        wu, bu = p['upd_W%d' % i], p['upd_b%d' % i]
        a_rows = h @ wm[:HID] + bm
        b_rows = h @ wm[HID:]
        agg = _edge_aggregate(a_rows, b_rows, dst)
        m = agg / deg[:, None]
        h = h + jnp.maximum(h @ wu[:HID] + m @ wu[HID:] + bu, 0.0)

    z = jnp.maximum(h @ p['proj_W0'] + p['proj_b0'], 0.0) @ p['proj_W1'] + p['proj_b1']
    return h, z


# speedup vs baseline: 8.3324x; 1.1734x over previous
import functools

import jax
import jax.numpy as jnp
from jax import lax
from jax.experimental import pallas as pl
from jax.experimental.pallas import tpu as pltpu
from jax.experimental.pallas import tpu_sc as plsc

B, N, NODE_DIM, HID, PROJ = 8, 2048, 5, 128, 256
NUM_MP = 3
BOX = 1.0
K_TOTAL = 48
NT = B * N


_NC, _NS = 2, 16
_NW = _NC * _NS
_NODES_W = NT // _NW
_GN = 8
_GE = _GN * K_TOTAL
_NGROUPS = _NODES_W // _GN
_CORE_NODES = NT // _NC


def _edge_body(a_hbm, b_hbm, dst_hbm, out_hbm, agg_sh, a_buf, rows, idx_buf,
               idx_loc, sem):
    c = lax.axis_index("c")
    s = lax.axis_index("s")
    wid = c * _NS + s
    core_base = c * _CORE_NODES

    def _zrow(i, _):
        for d in range(HID // 16):
            rows[i, pl.ds(d * 16, 16)] = jnp.zeros((16,), jnp.float32)
        return 0
    lax.fori_loop(0, _GE, _zrow, 0)
    sh_base = s * _NODES_W
    pltpu.sync_copy(rows, agg_sh.at[pl.ds(sh_base, _GE)])
    pltpu.sync_copy(rows.at[pl.ds(0, _NODES_W - _GE)],
                    agg_sh.at[pl.ds(sh_base + _GE, _NODES_W - _GE)])
    plsc.subcore_barrier()

    def _group(g, _):
        edge_base = wid * (_NODES_W * K_TOTAL) + g * _GE
        node_base = wid * _NODES_W + g * _GN
        pltpu.sync_copy(dst_hbm.at[pl.ds(edge_base, _GE)], idx_buf)
        pltpu.sync_copy(a_hbm.at[pl.ds(node_base, _GN)], a_buf)

        def _loc(i, _):
            v = idx_buf[pl.ds(i * 16, 16)]
            idx_loc[pl.ds(i * 16, 16)] = v - core_base
            return 0
        lax.fori_loop(0, _GE // 16, _loc, 0)

        pltpu.async_copy(b_hbm.at[idx_buf], rows, sem).wait()

        for n in range(_GN):
            a_regs = [a_buf[n, pl.ds(d * 16, 16)] for d in range(HID // 16)]

            def _edge(e, _):
                r = n * K_TOTAL + e
                for d in range(HID // 16):
                    v = rows[r, pl.ds(d * 16, 16)] + a_regs[d]
                    rows[r, pl.ds(d * 16, 16)] = jnp.maximum(v, 0.0)
                return 0
            lax.fori_loop(0, K_TOTAL, _edge, 0)

        pltpu.sync_copy(rows, agg_sh.at[idx_loc], add=True)
        return 0

    lax.fori_loop(0, _NGROUPS, _group, 0)
    plsc.subcore_barrier()

    pltpu.sync_copy(agg_sh.at[pl.ds(sh_base, _NODES_W)],
                    out_hbm.at[pl.ds(wid * _NODES_W, _NODES_W)])


@jax.jit
def _edge_aggregate(a_rows, b_rows, dst_flat):
    mesh = plsc.VectorSubcoreMesh(core_axis_name="c", subcore_axis_name="s")
    return pl.kernel(
        _edge_body,
        out_type=jax.ShapeDtypeStruct((NT, HID), jnp.float32),
        mesh=mesh,
        scratch_types=[
            pltpu.VMEM_SHARED((_CORE_NODES, HID), jnp.float32),
            pltpu.VMEM((_GN, HID), jnp.float32),
            pltpu.VMEM((_GE, HID), jnp.float32),
            pltpu.VMEM((_GE,), jnp.int32),
            pltpu.VMEM((_GE,), jnp.int32),
            pltpu.SemaphoreType.DMA,
        ],
    )(a_rows, b_rows, dst_flat)


def _encoder_body(x_ref, w0_ref, b0_ref, w1_ref, b1_ref, h_ref):
    x = x_ref[...]
    h = jnp.maximum(jnp.dot(x, w0_ref[...], preferred_element_type=jnp.float32)
                    + b0_ref[...], 0.0)
    h = jnp.maximum(jnp.dot(h, w1_ref[...], preferred_element_type=jnp.float32)
                    + b1_ref[...], 0.0)
    h_ref[...] = h


def _encode(x_pad, w0p, b0, w1, b1):
    blk = 2048
    grid = (NT // blk,)
    return pl.pallas_call(
        _encoder_body,
        grid=grid,
        in_specs=[
            pl.BlockSpec((blk, 8), lambda i: (i, 0)),
            pl.BlockSpec((8, HID), lambda i: (0, 0)),
            pl.BlockSpec((1, HID), lambda i: (0, 0)),
            pl.BlockSpec((HID, HID), lambda i: (0, 0)),
            pl.BlockSpec((1, HID), lambda i: (0, 0)),
        ],
        out_specs=pl.BlockSpec((blk, HID), lambda i: (i, 0)),
        out_shape=jax.ShapeDtypeStruct((NT, HID), jnp.float32),
    )(x_pad, w0p, b0, w1, b1)


_RB = 256
_NB = N // _RB
_W32 = N // 32


def _knn_body(pos_r_ref, pos_c_ref, q_r_ref, q_c_ref, pk_lo_ref, pk_hi_ref,
              bits_ref, deg_ref):
    nb = pl.program_id(1)
    acc = jnp.zeros((_RB, N), jnp.float32)
    for d in range(3):
        pr = pos_r_ref[0, :, d:d + 1]
        pc = pos_c_ref[0, d:d + 1, :]
        delta = jnp.abs(pr - pc)
        delta = jnp.where(delta > 0.5 * BOX, BOX - delta, delta)
        acc = acc + delta * delta
    dist = jnp.sqrt(acc + 1e-12)
    qr0, qr1 = q_r_ref[0, :, 0:1], q_r_ref[0, :, 1:2]
    qc0, qc1 = q_c_ref[0, 0:1, :], q_c_ref[0, 1:2, :]
    nr = jnp.maximum(jnp.sqrt(qr0 * qr0 + qr1 * qr1), 1e-8)
    nc = jnp.maximum(jnp.sqrt(qc0 * qc0 + qc1 * qc1), 1e-8)
    qdot = (qr0 * qc0 + qr1 * qc1) / (nr * nc)
    score = dist - 0.5 * (1.0 - qdot)
    col = lax.broadcasted_iota(jnp.int32, (_RB, N), 1)
    row = lax.broadcasted_iota(jnp.int32, (_RB, N), 0) + nb * _RB
    score = jnp.where(col == row, jnp.inf, score)
    k32 = lax.bitcast_convert_type(score, jnp.int32)
    key = jnp.where(k32 < 0, k32 ^ jnp.int32(0x7FFFFFFF), k32)
    ukey = lax.bitcast_convert_type(key, jnp.uint32) ^ jnp.uint32(0x80000000)
    lo0 = jnp.zeros((_RB, 1), jnp.uint32)
    hi0 = jnp.full((_RB, 1), 0xFFFFFFFF, jnp.uint32)

    def _bs(_, carry):
        lo, hi = carry
        mid = lo + ((hi - lo) >> jnp.uint32(1))
        cnt = jnp.sum((ukey <= mid).astype(jnp.float32), axis=1,
                      keepdims=True)
        ge = cnt >= float(K_TOTAL)
        return jnp.where(ge, lo, mid + jnp.uint32(1)), jnp.where(ge, mid, hi)

    lo, hi = lax.fori_loop(0, 32, _bs, (lo0, hi0))
    mask = (ukey <= hi).astype(jnp.float32)
    deg_part = jnp.sum(mask, axis=0).reshape(1, 1, N)

    @pl.when(nb == 0)
    def _():
        deg_ref[...] = deg_part

    @pl.when(nb != 0)
    def _():
        deg_ref[...] += deg_part

    lo = jnp.dot(mask, pk_lo_ref[...], preferred_element_type=jnp.float32)
    hi = jnp.dot(mask, pk_hi_ref[...], preferred_element_type=jnp.float32)
    bits_ref[0] = lo.astype(jnp.int32) | (hi.astype(jnp.int32) << 16)


@jax.jit
def _knn(pos_r, pos_c, q_r, q_c, pk_lo, pk_hi):
    grid = (B, _NB)
    bits, deg = pl.pallas_call(
        _knn_body,
        grid=grid,
        in_specs=[
            pl.BlockSpec((1, _RB, 3), lambda b, nb: (b, nb, 0)),
            pl.BlockSpec((1, 3, N), lambda b, nb: (b, 0, 0)),
            pl.BlockSpec((1, _RB, 2), lambda b, nb: (b, nb, 0)),
            pl.BlockSpec((1, 2, N), lambda b, nb: (b, 0, 0)),
            pl.BlockSpec((N, _W32), lambda b, nb: (0, 0)),
            pl.BlockSpec((N, _W32), lambda b, nb: (0, 0)),
        ],
        out_specs=[
            pl.BlockSpec((1, _RB, _W32), lambda b, nb: (b, nb, 0)),
            pl.BlockSpec((1, 1, N), lambda b, nb: (b, 0, 0)),
        ],
        out_shape=[
            jax.ShapeDtypeStruct((B, N, _W32), jnp.int32),
            jax.ShapeDtypeStruct((B, 1, N), jnp.float32),
        ],
    )(pos_r, pos_c, q_r, q_c, pk_lo, pk_hi)
    return bits, deg


_XROWS = 64
_XCH = _NODES_W // _XROWS
_OPAD = K_TOTAL


_PR = 4


def _extract_body(bits_hbm, dst_hbm, bits_buf, out_buf):
    c = lax.axis_index("c")
    s = lax.axis_index("s")
    wid = c * _NS + s
    b_off = (wid // 4) * N

    def _chunk(ch, _):
        row0 = wid * _NODES_W + ch * _XROWS
        pltpu.sync_copy(bits_hbm.at[pl.ds(row0, _XROWS)], bits_buf)

        def _row(r, _):
            def _wvec(wv, ptr):
                wvec = bits_buf[r, pl.ds(wv * 16, 16)]
                base = b_off + wv * 512
                for lane in range(16):
                    w0 = wvec[lane]
                    col0 = base + lane * 32
                    u = w0 & 0x7FFFFFFF
                    v = u - ((u >> 1) & 0x55555555)
                    v = (v & 0x33333333) + ((v >> 2) & 0x33333333)
                    v = (v + (v >> 4)) & 0x0F0F0F0F
                    pc = ((v * 0x01010101) >> 24) + ((w0 >> 31) & 1)

                    def _step(t, carry, col0=col0):
                        w, p = carry
                        lsb = w & (0 - w)
                        e = ((lax.bitcast_convert_type(
                            lsb.astype(jnp.float32), jnp.int32) >> 23)
                            & 0xFF) - 127
                        out_buf[pl.ds(p, 16)] = jnp.broadcast_to(
                            col0 + e, (16,)).astype(jnp.int32)
                        return (w ^ lsb, p + 1)

                    _, ptr = lax.fori_loop(0, pc, _step, (w0, ptr))
                return ptr

            lax.fori_loop(0, _W32 // 16, _wvec, r * _OPAD)
            return 0

        lax.fori_loop(0, _XROWS, _row, 0)
        pltpu.sync_copy(out_buf.at[pl.ds(0, _XROWS * _OPAD)],
                        dst_hbm.at[pl.ds(row0 * _OPAD, _XROWS * _OPAD)])
        return 0

    lax.fori_loop(0, _XCH, _chunk, 0)


@jax.jit
def _extract(bits):
    mesh = plsc.VectorSubcoreMesh(core_axis_name="c", subcore_axis_name="s")
    return pl.kernel(
        _extract_body,
        out_type=jax.ShapeDtypeStruct((NT * _OPAD,), jnp.int32),
        mesh=mesh,
        scratch_types=[
            pltpu.VMEM((_XROWS, _W32), jnp.int32),
            pltpu.VMEM((_XROWS * _OPAD + 16,), jnp.int32),
        ],
    )(bits)


def _build_edges(positions, q_values):
    pos_c = positions.transpose(0, 2, 1)
    q_c = q_values.transpose(0, 2, 1)
    j = jnp.arange(N)
    word = j[:, None] // 32 == jnp.arange(_W32)[None, :]
    wbit = j % 32
    weight = jnp.exp2((wbit % 16).astype(jnp.float32))[:, None]
    pk_lo = (word & (wbit < 16)[:, None]).astype(jnp.float32) * weight
    pk_hi = (word & (wbit >= 16)[:, None]).astype(jnp.float32) * weight
    bits, deg = _knn(positions, pos_c, q_values, q_c, pk_lo, pk_hi)
    dst = _extract(bits.reshape(NT, _W32))
    return dst, jnp.maximum(deg.reshape(NT), 1.0)


def kernel(x, positions, q_values, params, k_spatial, k_q):
    p = params
    x_pad = jnp.pad(x.reshape(NT, NODE_DIM), ((0, 0), (0, 8 - NODE_DIM)))
    w0p = jnp.pad(p['enc_W0'], ((0, 8 - NODE_DIM), (0, 0)))

    h = _encode(x_pad, w0p, p['enc_b0'][None, :], p['enc_W1'], p['enc_b1'][None, :])

    dst, deg = _build_edges(positions, q_values)

    for i in range(NUM_MP):
        wm, bm = p['msg_W%d' % i], p['msg_b%d' % i]
        wu, bu = p['upd_W%d' % i], p['upd_b%d' % i]
        a_rows = h @ wm[:HID] + bm
        b_rows = h @ wm[HID:]
        agg = _edge_aggregate(a_rows, b_rows, dst)
        m = agg / deg[:, None]
        h = h + jnp.maximum(h @ wu[:HID] + m @ wu[HID:] + bu, 0.0)

    z = jnp.maximum(h @ p['proj_W0'] + p['proj_b0'], 0.0) @ p['proj_W1'] + p['proj_b1']
    return h, z
